# Initial kernel scaffold; baseline (speedup 1.0000x reference)
import jax
import jax.numpy as jnp
from jax.experimental import pallas as pl


def kernel(features, edge_index, W1, b1, W2, b2):
    raise NotImplementedError("write your pallas kernel here")


# speedup vs baseline: 10.8715x; 10.8715x over previous
import functools

import jax
import jax.numpy as jnp
from jax import lax
from jax.experimental import pallas as pl
from jax.experimental.pallas import tpu as pltpu
from jax.experimental.pallas import tpu_sc as plsc

N_NODES = 10000
N_EDGES = 320000
F_IN = 128
HID = 16
OUT = 40

NC = 2
NS = 16
NW = NC * NS
LANES = 16
CHUNK = 128
EDGES_PER_ROUND = NW * CHUNK
K = -(-N_EDGES // EDGES_PER_ROUND)
E_PAD = K * EDGES_PER_ROUND
NPAD = 10112
RPS = NPAD // NS

def _deg_body(srcs_hbm, dsts_hbm, ones_hbm, zeros_hbm, out_hbm,
              sidx_v, didx_v, ones_v, ds_sh, dd_sh):
    c = lax.axis_index("c")
    s = lax.axis_index("s")
    wid = c * NS + s
    r0 = s * RPS
    pltpu.sync_copy(zeros_hbm.at[pl.ds(r0, RPS)], ds_sh.at[pl.ds(r0, RPS)])
    pltpu.sync_copy(zeros_hbm.at[pl.ds(r0, RPS)], dd_sh.at[pl.ds(r0, RPS)])
    pltpu.sync_copy(ones_hbm, ones_v)
    pltpu.sync_copy(srcs_hbm.at[wid], sidx_v)
    pltpu.sync_copy(dsts_hbm.at[wid], didx_v)
    plsc.subcore_barrier()

    @pl.loop(0, K)
    def _(j):
        pltpu.sync_copy(ones_v, ds_sh.at[sidx_v.at[j]], add=True)
        pltpu.sync_copy(ones_v, dd_sh.at[didx_v.at[j]], add=True)

    plsc.subcore_barrier()
    pltpu.sync_copy(ds_sh.at[pl.ds(r0, RPS)], out_hbm.at[c, 0, pl.ds(r0, RPS)])
    pltpu.sync_copy(dd_sh.at[pl.ds(r0, RPS)], out_hbm.at[c, 1, pl.ds(r0, RPS)])


def _agg_body(h_hbm, srcs_hbm, dsts_hbm, zeros_hbm, out_hbm,
              sidx_v, didx_v, rows_v, acc_sh, sem):
    c = lax.axis_index("c")
    s = lax.axis_index("s")
    wid = c * NS + s
    r0 = s * RPS
    pltpu.sync_copy(zeros_hbm.at[pl.ds(r0, RPS)], acc_sh.at[pl.ds(r0, RPS)])
    pltpu.sync_copy(srcs_hbm.at[wid], sidx_v)
    pltpu.sync_copy(dsts_hbm.at[wid], didx_v)
    plsc.subcore_barrier()

    @pl.loop(0, K)
    def _(j):
        pltpu.async_copy(h_hbm.at[sidx_v.at[j]], rows_v, sem).wait()
        pltpu.sync_copy(rows_v, acc_sh.at[didx_v.at[j]], add=True)

    plsc.subcore_barrier()
    pltpu.sync_copy(acc_sh.at[pl.ds(r0, RPS)], out_hbm.at[c, pl.ds(r0, RPS)])


def _tc1_body(feat_ref, w1_ref, degs_ref, out_ref):
    degs = degs_ref[...]
    deg_src = degs[0, 0] + degs[1, 0]
    norm_src = lax.rsqrt(jnp.maximum(deg_src[:, 0:1], 1.0))
    h = jnp.dot(feat_ref[...], w1_ref[...], preferred_element_type=jnp.float32)
    out_ref[...] = h * norm_src


def _tc2_body(aggs_ref, degs_ref, b1_ref, out_ref):
    degs = degs_ref[...]
    norm_src = lax.rsqrt(jnp.maximum((degs[0, 0] + degs[1, 0])[:, 0:1], 1.0))
    norm_dst = lax.rsqrt(jnp.maximum((degs[0, 1] + degs[1, 1])[:, 0:1], 1.0))
    aggs = aggs_ref[...]
    agg = aggs[0] + aggs[1]
    x = jnp.maximum(agg * norm_dst + b1_ref[...], 0.0) * norm_src
    row = lax.broadcasted_iota(jnp.int32, (NPAD, 1), 0)
    out_ref[...] = jnp.where(row < N_NODES, x, 0.0)


def _tc3_body(aggs_ref, degs_ref, w2_ref, b2_ref, out_ref):
    degs = degs_ref[...]
    norm_dst = lax.rsqrt(
        jnp.maximum((degs[0, 1] + degs[1, 1])[:N_NODES, 0:1], 1.0))
    aggs = aggs_ref[...]
    agg = aggs[0, :N_NODES] + aggs[1, :N_NODES]
    rst = jnp.dot(agg * norm_dst, w2_ref[...],
                  preferred_element_type=jnp.float32)
    out_ref[...] = rst + b2_ref[...]


@functools.cache
def _sc_kernels():
    mesh = plsc.VectorSubcoreMesh(core_axis_name="c", subcore_axis_name="s",
                                  num_cores=NC, num_subcores=NS)
    cp = pltpu.CompilerParams(use_tc_tiling_on_sc=False)
    deg = pl.kernel(
        _deg_body,
        out_type=jax.ShapeDtypeStruct((NC, 2, NPAD, LANES), jnp.float32),
        mesh=mesh,
        scratch_types=[
            pltpu.VMEM((K, CHUNK), jnp.int32),
            pltpu.VMEM((K, CHUNK), jnp.int32),
            pltpu.VMEM((CHUNK, LANES), jnp.float32),
            pltpu.VMEM_SHARED((NPAD, LANES), jnp.float32),
            pltpu.VMEM_SHARED((NPAD, LANES), jnp.float32),
        ],
        compiler_params=cp,
    )
    agg = pl.kernel(
        _agg_body,
        out_type=jax.ShapeDtypeStruct((NC, NPAD, LANES), jnp.float32),
        mesh=mesh,
        scratch_types=[
            pltpu.VMEM((K, CHUNK), jnp.int32),
            pltpu.VMEM((K, CHUNK), jnp.int32),
            pltpu.VMEM((CHUNK, LANES), jnp.float32),
            pltpu.VMEM_SHARED((NPAD, LANES), jnp.float32),
            pltpu.SemaphoreType.DMA,
        ],
        compiler_params=cp,
    )
    return deg, agg


_tc1 = pl.pallas_call(
    _tc1_body, out_shape=jax.ShapeDtypeStruct((NPAD, HID), jnp.float32))
_tc2 = pl.pallas_call(
    _tc2_body, out_shape=jax.ShapeDtypeStruct((NPAD, HID), jnp.float32))
_tc3 = pl.pallas_call(
    _tc3_body, out_shape=jax.ShapeDtypeStruct((N_NODES, OUT), jnp.float32))


def kernel(features, edge_index, W1, b1, W2, b2):
    pad = E_PAD - N_EDGES
    fill = jnp.full((pad,), N_NODES, jnp.int32)
    srcs = jnp.concatenate([edge_index[0], fill]).reshape(NW, K, CHUNK)
    dsts = jnp.concatenate([edge_index[1], fill]).reshape(NW, K, CHUNK)
    feat_pad = jnp.concatenate(
        [features, jnp.zeros((NPAD - N_NODES, F_IN), jnp.float32)])
    ones_rows = jnp.ones((CHUNK, LANES), jnp.float32)
    zeros_rows = jnp.zeros((NPAD, LANES), jnp.float32)

    deg_kernel, agg_kernel = _sc_kernels()
    degs = deg_kernel(srcs, dsts, ones_rows, zeros_rows)
    h1 = _tc1(feat_pad, W1, degs)
    agg1 = agg_kernel(h1, srcs, dsts, zeros_rows)
    x2 = _tc2(agg1, degs, b1.reshape(1, HID))
    agg2 = agg_kernel(x2, srcs, dsts, zeros_rows)
    return _tc3(agg2, degs, W2, b2.reshape(1, OUT))


# speedup vs baseline: 12.4185x; 1.1423x over previous
import functools

import jax
import jax.numpy as jnp
from jax import lax
from jax.experimental import pallas as pl
from jax.experimental.pallas import tpu as pltpu
from jax.experimental.pallas import tpu_sc as plsc

N_NODES = 10000
N_EDGES = 320000
F_IN = 128
HID = 16
OUT = 40

NC = 2
NS = 16
NW = NC * NS
LANES = 16
CHUNK = 128
NBUF = 8
EDGES_PER_ROUND = NW * CHUNK
K = 80
E_PAD = K * EDGES_PER_ROUND
NPAD = 10112
RPS = NPAD // NS


def _deg_body(srcs_hbm, dsts_hbm, ones_hbm, zeros_hbm, out_hbm,
              sidx_v, didx_v, ones_v, ds_sh, dd_sh, ssem, dsem):
    c = lax.axis_index("c")
    s = lax.axis_index("s")
    wid = c * NS + s
    r0 = s * RPS
    pltpu.sync_copy(zeros_hbm.at[pl.ds(r0, RPS)], ds_sh.at[pl.ds(r0, RPS)])
    pltpu.sync_copy(zeros_hbm.at[pl.ds(r0, RPS)], dd_sh.at[pl.ds(r0, RPS)])
    pltpu.sync_copy(ones_hbm, ones_v)
    pltpu.sync_copy(srcs_hbm.at[wid], sidx_v)
    pltpu.sync_copy(dsts_hbm.at[wid], didx_v)
    plsc.subcore_barrier()

    @pl.loop(0, K)
    def _(j):
        pltpu.async_copy(ones_v, ds_sh.at[sidx_v.at[j]], ssem, add=True)
        pltpu.async_copy(ones_v, dd_sh.at[didx_v.at[j]], dsem, add=True)

    @pl.loop(0, K)
    def _(j):
        pltpu.make_async_copy(ones_v, ds_sh.at[sidx_v.at[j]], ssem).wait()
        pltpu.make_async_copy(ones_v, dd_sh.at[didx_v.at[j]], dsem).wait()

    plsc.subcore_barrier()
    pltpu.sync_copy(ds_sh.at[pl.ds(r0, RPS)], out_hbm.at[c, 0, pl.ds(r0, RPS)])
    pltpu.sync_copy(dd_sh.at[pl.ds(r0, RPS)], out_hbm.at[c, 1, pl.ds(r0, RPS)])


def _agg_body(h_hbm, srcs_hbm, dsts_hbm, zeros_hbm, out_hbm,
              sidx_v, didx_v, rows_v, acc_sh, gsem, ssem):
    c = lax.axis_index("c")
    s = lax.axis_index("s")
    wid = c * NS + s
    r0 = s * RPS
    pltpu.sync_copy(zeros_hbm.at[pl.ds(r0, RPS)], acc_sh.at[pl.ds(r0, RPS)])
    pltpu.sync_copy(srcs_hbm.at[wid], sidx_v)
    pltpu.sync_copy(dsts_hbm.at[wid], didx_v)
    plsc.subcore_barrier()

    for b in range(NBUF):
        pltpu.async_copy(h_hbm.at[sidx_v.at[b]], rows_v.at[b], gsem.at[b])

    @pl.loop(0, K // NBUF)
    def _(i):
        j0 = i * NBUF
        for b in range(NBUF):
            pltpu.make_async_copy(
                h_hbm.at[sidx_v.at[j0 + b]], rows_v.at[b], gsem.at[b]).wait()
            pltpu.async_copy(
                rows_v.at[b], acc_sh.at[didx_v.at[j0 + b]], ssem.at[b],
                add=True)

        @pl.when(i < K // NBUF - 1)
        def _():
            for b in range(NBUF):
                pltpu.make_async_copy(
                    rows_v.at[b], acc_sh.at[didx_v.at[j0 + b]],
                    ssem.at[b]).wait()
                pltpu.async_copy(
                    h_hbm.at[sidx_v.at[j0 + NBUF + b]], rows_v.at[b],
                    gsem.at[b])

    for b in range(NBUF):
        pltpu.make_async_copy(
            rows_v.at[b], acc_sh.at[didx_v.at[K - NBUF + b]], ssem.at[b]).wait()

    plsc.subcore_barrier()
    pltpu.sync_copy(acc_sh.at[pl.ds(r0, RPS)], out_hbm.at[c, pl.ds(r0, RPS)])


def _tc_mm_body(feat_ref, w1_ref, out_ref):
    out_ref[...] = jnp.dot(feat_ref[...], w1_ref[...],
                           preferred_element_type=jnp.float32)


def _tc_scale_body(h_ref, degs_ref, out_ref):
    degs = degs_ref[...]
    deg_src = degs[0, 0] + degs[1, 0]
    norm_src = lax.rsqrt(jnp.maximum(deg_src[:, 0:1], 1.0))
    out_ref[...] = h_ref[...] * norm_src


def _tc2_body(aggs_ref, degs_ref, b1_ref, out_ref):
    degs = degs_ref[...]
    norm_src = lax.rsqrt(jnp.maximum((degs[0, 0] + degs[1, 0])[:, 0:1], 1.0))
    norm_dst = lax.rsqrt(jnp.maximum((degs[0, 1] + degs[1, 1])[:, 0:1], 1.0))
    aggs = aggs_ref[...]
    agg = aggs[0] + aggs[1]
    x = jnp.maximum(agg * norm_dst + b1_ref[...], 0.0) * norm_src
    row = lax.broadcasted_iota(jnp.int32, (NPAD, 1), 0)
    out_ref[...] = jnp.where(row < N_NODES, x, 0.0)


def _tc3_body(aggs_ref, degs_ref, w2_ref, b2_ref, out_ref):
    degs = degs_ref[...]
    norm_dst = lax.rsqrt(
        jnp.maximum((degs[0, 1] + degs[1, 1])[:N_NODES, 0:1], 1.0))
    aggs = aggs_ref[...]
    agg = aggs[0, :N_NODES] + aggs[1, :N_NODES]
    rst = jnp.dot(agg * norm_dst, w2_ref[...],
                  preferred_element_type=jnp.float32)
    out_ref[...] = rst + b2_ref[...]


@functools.cache
def _sc_kernels():
    mesh = plsc.VectorSubcoreMesh(core_axis_name="c", subcore_axis_name="s",
                                  num_cores=NC, num_subcores=NS)
    cp = pltpu.CompilerParams(use_tc_tiling_on_sc=False)
    deg = pl.kernel(
        _deg_body,
        out_type=jax.ShapeDtypeStruct((NC, 2, NPAD, LANES), jnp.float32),
        mesh=mesh,
        scratch_types=[
            pltpu.VMEM((K, CHUNK), jnp.int32),
            pltpu.VMEM((K, CHUNK), jnp.int32),
            pltpu.VMEM((CHUNK, LANES), jnp.float32),
            pltpu.VMEM_SHARED((NPAD, LANES), jnp.float32),
            pltpu.VMEM_SHARED((NPAD, LANES), jnp.float32),
            pltpu.SemaphoreType.DMA,
            pltpu.SemaphoreType.DMA,
        ],
        compiler_params=cp,
    )
    agg = pl.kernel(
        _agg_body,
        out_type=jax.ShapeDtypeStruct((NC, NPAD, LANES), jnp.float32),
        mesh=mesh,
        scratch_types=[
            pltpu.VMEM((K, CHUNK), jnp.int32),
            pltpu.VMEM((K, CHUNK), jnp.int32),
            pltpu.VMEM((NBUF, CHUNK, LANES), jnp.float32),
            pltpu.VMEM_SHARED((NPAD, LANES), jnp.float32),
            pltpu.SemaphoreType.DMA((NBUF,)),
            pltpu.SemaphoreType.DMA((NBUF,)),
        ],
        compiler_params=cp,
    )
    return deg, agg


_tc_mm = pl.pallas_call(
    _tc_mm_body, out_shape=jax.ShapeDtypeStruct((NPAD, HID), jnp.float32))
_tc_scale = pl.pallas_call(
    _tc_scale_body, out_shape=jax.ShapeDtypeStruct((NPAD, HID), jnp.float32))
_tc2 = pl.pallas_call(
    _tc2_body, out_shape=jax.ShapeDtypeStruct((NPAD, HID), jnp.float32))
_tc3 = pl.pallas_call(
    _tc3_body, out_shape=jax.ShapeDtypeStruct((N_NODES, OUT), jnp.float32))


def kernel(features, edge_index, W1, b1, W2, b2):
    pad = E_PAD - N_EDGES
    fill = jnp.full((pad,), N_NODES, jnp.int32)
    srcs = jnp.concatenate([edge_index[0], fill]).reshape(NW, K, CHUNK)
    dsts = jnp.concatenate([edge_index[1], fill]).reshape(NW, K, CHUNK)
    feat_pad = jnp.concatenate(
        [features, jnp.zeros((NPAD - N_NODES, F_IN), jnp.float32)])
    ones_rows = jnp.ones((CHUNK, LANES), jnp.float32)
    zeros_rows = jnp.zeros((NPAD, LANES), jnp.float32)

    deg_kernel, agg_kernel = _sc_kernels()
    degs = deg_kernel(srcs, dsts, ones_rows, zeros_rows)
    h1p = _tc_mm(feat_pad, W1)
    h1 = _tc_scale(h1p, degs)
    agg1 = agg_kernel(h1, srcs, dsts, zeros_rows)
    x2 = _tc2(agg1, degs, b1.reshape(1, HID))
    agg2 = agg_kernel(x2, srcs, dsts, zeros_rows)
    return _tc3(agg2, degs, W2, b2.reshape(1, OUT))


# speedup vs baseline: 16.7180x; 1.3462x over previous
import functools

import jax
import jax.numpy as jnp
from jax import lax
from jax.experimental import pallas as pl
from jax.experimental.pallas import tpu as pltpu
from jax.experimental.pallas import tpu_sc as plsc

N_NODES = 10000
N_EDGES = 320000
F_IN = 128
HID = 16
OUT = 40

NC = 2
NS = 16
NW = NC * NS
LANES = 16
CHUNK = 128
NBUF = 8
EDGES_PER_ROUND = NW * CHUNK
K = 80
E_PAD = K * EDGES_PER_ROUND
NPAD = 10112
RPS = NPAD // NS


def _deg_body(srcs_hbm, dsts_hbm, ones_hbm, zeros_hbm, out_hbm,
              sidx_v, didx_v, ones_v, ds_sh, dd_sh, ssem, dsem):
    c = lax.axis_index("c")
    s = lax.axis_index("s")
    wid = c * NS + s
    r0 = s * RPS
    pltpu.sync_copy(zeros_hbm.at[pl.ds(r0, RPS)], ds_sh.at[pl.ds(r0, RPS)])
    pltpu.sync_copy(zeros_hbm.at[pl.ds(r0, RPS)], dd_sh.at[pl.ds(r0, RPS)])
    pltpu.sync_copy(ones_hbm, ones_v)
    pltpu.sync_copy(srcs_hbm.at[wid], sidx_v)
    pltpu.sync_copy(dsts_hbm.at[wid], didx_v)
    plsc.subcore_barrier()

    @pl.loop(0, K)
    def _(j):
        pltpu.async_copy(ones_v, ds_sh.at[sidx_v.at[j]], ssem, add=True)
        pltpu.async_copy(ones_v, dd_sh.at[didx_v.at[j]], dsem, add=True)

    @pl.loop(0, K)
    def _(j):
        pltpu.make_async_copy(ones_v, ds_sh.at[sidx_v.at[j]], ssem).wait()
        pltpu.make_async_copy(ones_v, dd_sh.at[didx_v.at[j]], dsem).wait()

    plsc.subcore_barrier()
    pltpu.sync_copy(ds_sh.at[pl.ds(r0, RPS)], out_hbm.at[c, 0, pl.ds(r0, RPS)])
    pltpu.sync_copy(dd_sh.at[pl.ds(r0, RPS)], out_hbm.at[c, 1, pl.ds(r0, RPS)])


def _agg_body(h_hbm, srcs_hbm, dsts_hbm, zeros_hbm, out_hbm,
              sidx_v, didx_v, rows_v, h_sh, acc_sh, gsem, ssem):
    c = lax.axis_index("c")
    s = lax.axis_index("s")
    wid = c * NS + s
    r0 = s * RPS
    pltpu.sync_copy(zeros_hbm.at[pl.ds(r0, RPS)], acc_sh.at[pl.ds(r0, RPS)])
    pltpu.sync_copy(h_hbm.at[pl.ds(r0, RPS)], h_sh.at[pl.ds(r0, RPS)])
    pltpu.sync_copy(srcs_hbm.at[wid], sidx_v)
    pltpu.sync_copy(dsts_hbm.at[wid], didx_v)
    plsc.subcore_barrier()

    for b in range(NBUF):
        pltpu.async_copy(h_sh.at[sidx_v.at[b]], rows_v.at[b], gsem.at[b])

    @pl.loop(0, K // NBUF)
    def _(i):
        j0 = i * NBUF
        for b in range(NBUF):
            pltpu.make_async_copy(
                h_sh.at[sidx_v.at[j0 + b]], rows_v.at[b], gsem.at[b]).wait()
            pltpu.async_copy(
                rows_v.at[b], acc_sh.at[didx_v.at[j0 + b]], ssem.at[b],
                add=True)

        @pl.when(i < K // NBUF - 1)
        def _():
            for b in range(NBUF):
                pltpu.make_async_copy(
                    rows_v.at[b], acc_sh.at[didx_v.at[j0 + b]],
                    ssem.at[b]).wait()
                pltpu.async_copy(
                    h_sh.at[sidx_v.at[j0 + NBUF + b]], rows_v.at[b],
                    gsem.at[b])

    for b in range(NBUF):
        pltpu.make_async_copy(
            rows_v.at[b], acc_sh.at[didx_v.at[K - NBUF + b]], ssem.at[b]).wait()

    plsc.subcore_barrier()
    pltpu.sync_copy(acc_sh.at[pl.ds(r0, RPS)], out_hbm.at[c, pl.ds(r0, RPS)])


def _tc_mm_body(feat_ref, w1_ref, out_ref):
    out_ref[...] = jnp.dot(feat_ref[...], w1_ref[...],
                           preferred_element_type=jnp.float32)


def _tc_scale_body(h_ref, degs_ref, out_ref):
    degs = degs_ref[...]
    deg_src = degs[0, 0] + degs[1, 0]
    norm_src = lax.rsqrt(jnp.maximum(deg_src[:, 0:1], 1.0))
    out_ref[...] = h_ref[...] * norm_src


def _tc2_body(aggs_ref, degs_ref, b1_ref, out_ref):
    degs = degs_ref[...]
    norm_src = lax.rsqrt(jnp.maximum((degs[0, 0] + degs[1, 0])[:, 0:1], 1.0))
    norm_dst = lax.rsqrt(jnp.maximum((degs[0, 1] + degs[1, 1])[:, 0:1], 1.0))
    aggs = aggs_ref[...]
    agg = aggs[0] + aggs[1]
    x = jnp.maximum(agg * norm_dst + b1_ref[...], 0.0) * norm_src
    row = lax.broadcasted_iota(jnp.int32, (NPAD, 1), 0)
    out_ref[...] = jnp.where(row < N_NODES, x, 0.0)


def _tc3_body(aggs_ref, degs_ref, w2_ref, b2_ref, out_ref):
    degs = degs_ref[...]
    norm_dst = lax.rsqrt(
        jnp.maximum((degs[0, 1] + degs[1, 1])[:N_NODES, 0:1], 1.0))
    aggs = aggs_ref[...]
    agg = aggs[0, :N_NODES] + aggs[1, :N_NODES]
    rst = jnp.dot(agg * norm_dst, w2_ref[...],
                  preferred_element_type=jnp.float32)
    out_ref[...] = rst + b2_ref[...]


@functools.cache
def _sc_kernels():
    mesh = plsc.VectorSubcoreMesh(core_axis_name="c", subcore_axis_name="s",
                                  num_cores=NC, num_subcores=NS)
    cp = pltpu.CompilerParams(use_tc_tiling_on_sc=False)
    deg = pl.kernel(
        _deg_body,
        out_type=jax.ShapeDtypeStruct((NC, 2, NPAD, LANES), jnp.float32),
        mesh=mesh,
        scratch_types=[
            pltpu.VMEM((K, CHUNK), jnp.int32),
            pltpu.VMEM((K, CHUNK), jnp.int32),
            pltpu.VMEM((CHUNK, LANES), jnp.float32),
            pltpu.VMEM_SHARED((NPAD, LANES), jnp.float32),
            pltpu.VMEM_SHARED((NPAD, LANES), jnp.float32),
            pltpu.SemaphoreType.DMA,
            pltpu.SemaphoreType.DMA,
        ],
        compiler_params=cp,
    )
    agg = pl.kernel(
        _agg_body,
        out_type=jax.ShapeDtypeStruct((NC, NPAD, LANES), jnp.float32),
        mesh=mesh,
        scratch_types=[
            pltpu.VMEM((K, CHUNK), jnp.int32),
            pltpu.VMEM((K, CHUNK), jnp.int32),
            pltpu.VMEM((NBUF, CHUNK, LANES), jnp.float32),
            pltpu.VMEM_SHARED((NPAD, LANES), jnp.float32),
            pltpu.VMEM_SHARED((NPAD, LANES), jnp.float32),
            pltpu.SemaphoreType.DMA((NBUF,)),
            pltpu.SemaphoreType.DMA((NBUF,)),
        ],
        compiler_params=cp,
    )
    return deg, agg


_tc_mm = pl.pallas_call(
    _tc_mm_body, out_shape=jax.ShapeDtypeStruct((NPAD, HID), jnp.float32))
_tc_scale = pl.pallas_call(
    _tc_scale_body, out_shape=jax.ShapeDtypeStruct((NPAD, HID), jnp.float32))
_tc2 = pl.pallas_call(
    _tc2_body, out_shape=jax.ShapeDtypeStruct((NPAD, HID), jnp.float32))
_tc3 = pl.pallas_call(
    _tc3_body, out_shape=jax.ShapeDtypeStruct((N_NODES, OUT), jnp.float32))


def kernel(features, edge_index, W1, b1, W2, b2):
    pad = E_PAD - N_EDGES
    fill = jnp.full((pad,), N_NODES, jnp.int32)
    srcs = jnp.concatenate([edge_index[0], fill]).reshape(NW, K, CHUNK)
    dsts = jnp.concatenate([edge_index[1], fill]).reshape(NW, K, CHUNK)
    feat_pad = jnp.concatenate(
        [features, jnp.zeros((NPAD - N_NODES, F_IN), jnp.float32)])
    ones_rows = jnp.ones((CHUNK, LANES), jnp.float32)
    zeros_rows = jnp.zeros((NPAD, LANES), jnp.float32)

    deg_kernel, agg_kernel = _sc_kernels()
    degs = deg_kernel(srcs, dsts, ones_rows, zeros_rows)
    h1p = _tc_mm(feat_pad, W1)
    h1 = _tc_scale(h1p, degs)
    agg1 = agg_kernel(h1, srcs, dsts, zeros_rows)
    x2 = _tc2(agg1, degs, b1.reshape(1, HID))
    agg2 = agg_kernel(x2, srcs, dsts, zeros_rows)
    return _tc3(agg2, degs, W2, b2.reshape(1, OUT))


# speedup vs baseline: 26.4199x; 1.5803x over previous
import functools

import jax
import jax.numpy as jnp
from jax import lax
from jax.experimental import pallas as pl
from jax.experimental.pallas import tpu as pltpu
from jax.experimental.pallas import tpu_sc as plsc

N_NODES = 10000
N_EDGES = 320000
F_IN = 128
HID = 16
OUT = 40

NC = 2
NS = 16
NW = NC * NS
LANES = 16
CHUNK = 80
EPT = N_EDGES // NW
K = EPT // CHUNK
NBUF = 5
NPAD = 10112
RPS = NPAD // NS
NROWS = NPAD // 8
RPF = NROWS // NS


def _repack_to_flat(buf16, buf128):
    @pl.loop(0, RPF)
    def _(r):
        for m in range(8):
            buf128.at[r, pl.ds(16 * m, 16)][...] = buf16.at[8 * r + m][...]


def _repack_from_flat(buf128, buf16):
    @pl.loop(0, RPF)
    def _(r):
        for m in range(8):
            buf16.at[8 * r + m][...] = buf128.at[r, pl.ds(16 * m, 16)][...]


def _deg_body(srcs_hbm, dsts_hbm, ones_hbm, zeros_hbm, out_hbm,
              sidx_v, didx_v, ones_v, buf16, buf128, ds_sh, dd_sh,
              ssem, dsem):
    c = lax.axis_index("c")
    s = lax.axis_index("s")
    wid = c * NS + s
    r0 = s * RPS
    f0 = s * RPF
    pltpu.sync_copy(zeros_hbm.at[pl.ds(r0, RPS)], ds_sh.at[pl.ds(r0, RPS)])
    pltpu.sync_copy(zeros_hbm.at[pl.ds(r0, RPS)], dd_sh.at[pl.ds(r0, RPS)])
    pltpu.sync_copy(ones_hbm, ones_v)
    pltpu.sync_copy(srcs_hbm.at[wid], sidx_v)
    pltpu.sync_copy(dsts_hbm.at[wid], didx_v)
    plsc.subcore_barrier()

    @pl.loop(0, K)
    def _(j):
        pltpu.async_copy(ones_v, ds_sh.at[sidx_v.at[j]], ssem, add=True)
        pltpu.async_copy(ones_v, dd_sh.at[didx_v.at[j]], dsem, add=True)

    @pl.loop(0, K)
    def _(j):
        pltpu.make_async_copy(ones_v, ds_sh.at[sidx_v.at[j]], ssem).wait()
        pltpu.make_async_copy(ones_v, dd_sh.at[didx_v.at[j]], dsem).wait()

    plsc.subcore_barrier()
    pltpu.sync_copy(ds_sh.at[pl.ds(r0, RPS)], buf16)
    _repack_to_flat(buf16, buf128)
    pltpu.sync_copy(buf128, out_hbm.at[c, 0, pl.ds(f0, RPF)])
    pltpu.sync_copy(dd_sh.at[pl.ds(r0, RPS)], buf16)
    _repack_to_flat(buf16, buf128)
    pltpu.sync_copy(buf128, out_hbm.at[c, 1, pl.ds(f0, RPF)])


def _agg_body(h_hbm, srcs_hbm, dsts_hbm, zeros_hbm, out_hbm,
              sidx_v, didx_v, rows_v, buf16, buf128, h_sh, acc_sh,
              gsem, ssem):
    c = lax.axis_index("c")
    s = lax.axis_index("s")
    wid = c * NS + s
    r0 = s * RPS
    f0 = s * RPF
    pltpu.sync_copy(zeros_hbm.at[pl.ds(r0, RPS)], acc_sh.at[pl.ds(r0, RPS)])
    pltpu.sync_copy(h_hbm.at[pl.ds(f0, RPF)], buf128)
    _repack_from_flat(buf128, buf16)
    pltpu.sync_copy(buf16, h_sh.at[pl.ds(r0, RPS)])
    pltpu.sync_copy(srcs_hbm.at[wid], sidx_v)
    pltpu.sync_copy(dsts_hbm.at[wid], didx_v)
    plsc.subcore_barrier()

    for b in range(NBUF):
        pltpu.async_copy(h_sh.at[sidx_v.at[b]], rows_v.at[b], gsem.at[b])

    @pl.loop(0, K // NBUF)
    def _(i):
        j0 = i * NBUF
        for b in range(NBUF):
            pltpu.make_async_copy(
                h_sh.at[sidx_v.at[j0 + b]], rows_v.at[b], gsem.at[b]).wait()
            pltpu.async_copy(
                rows_v.at[b], acc_sh.at[didx_v.at[j0 + b]], ssem.at[b],
                add=True)

        @pl.when(i < K // NBUF - 1)
        def _():
            for b in range(NBUF):
                pltpu.make_async_copy(
                    rows_v.at[b], acc_sh.at[didx_v.at[j0 + b]],
                    ssem.at[b]).wait()
                pltpu.async_copy(
                    h_sh.at[sidx_v.at[j0 + NBUF + b]], rows_v.at[b],
                    gsem.at[b])

    for b in range(NBUF):
        pltpu.make_async_copy(
            rows_v.at[b], acc_sh.at[didx_v.at[K - NBUF + b]], ssem.at[b]).wait()

    plsc.subcore_barrier()
    pltpu.sync_copy(acc_sh.at[pl.ds(r0, RPS)], buf16)
    _repack_to_flat(buf16, buf128)
    pltpu.sync_copy(buf128, out_hbm.at[c, pl.ds(f0, RPF)])


def _tc_mm_body(feat_ref, wk1_ref, out_ref):
    out_ref[...] = jnp.dot(feat_ref[...], wk1_ref[...],
                           preferred_element_type=jnp.float32)


def _tc_scale_body(h_ref, degs_ref, out_ref):
    degs = degs_ref[...]
    norm_src = lax.rsqrt(jnp.maximum(degs[0, 0] + degs[1, 0], 1.0))
    out_ref[...] = h_ref[...] * norm_src


def _tc2_body(aggs_ref, degs_ref, b1_ref, out_ref):
    degs = degs_ref[...]
    norm_src = lax.rsqrt(jnp.maximum(degs[0, 0] + degs[1, 0], 1.0))
    norm_dst = lax.rsqrt(jnp.maximum(degs[0, 1] + degs[1, 1], 1.0))
    aggs = aggs_ref[...]
    agg = aggs[0] + aggs[1]
    out_ref[...] = jnp.maximum(agg * norm_dst + b1_ref[...], 0.0) * norm_src


def _tc3_body(aggs_ref, degs_ref, wk2_ref, b2_ref, out_ref):
    degs = degs_ref[...]
    norm_dst = lax.rsqrt(jnp.maximum(degs[0, 1] + degs[1, 1], 1.0))
    aggs = aggs_ref[...]
    scaled = (aggs[0] + aggs[1]) * norm_dst
    rst = jnp.dot(scaled, wk2_ref[...], preferred_element_type=jnp.float32)
    out_ref[...] = rst + b2_ref[...]


@functools.cache
def _sc_kernels():
    mesh = plsc.VectorSubcoreMesh(core_axis_name="c", subcore_axis_name="s",
                                  num_cores=NC, num_subcores=NS)
    cp = pltpu.CompilerParams(use_tc_tiling_on_sc=False)
    deg = pl.kernel(
        _deg_body,
        out_type=jax.ShapeDtypeStruct((NC, 2, NROWS, 128), jnp.float32),
        mesh=mesh,
        scratch_types=[
            pltpu.VMEM((K, CHUNK), jnp.int32),
            pltpu.VMEM((K, CHUNK), jnp.int32),
            pltpu.VMEM((CHUNK, LANES), jnp.float32),
            pltpu.VMEM((RPS, LANES), jnp.float32),
            pltpu.VMEM((RPF, 128), jnp.float32),
            pltpu.VMEM_SHARED((NPAD, LANES), jnp.float32),
            pltpu.VMEM_SHARED((NPAD, LANES), jnp.float32),
            pltpu.SemaphoreType.DMA,
            pltpu.SemaphoreType.DMA,
        ],
        compiler_params=cp,
    )
    agg = pl.kernel(
        _agg_body,
        out_type=jax.ShapeDtypeStruct((NC, NROWS, 128), jnp.float32),
        mesh=mesh,
        scratch_types=[
            pltpu.VMEM((K, CHUNK), jnp.int32),
            pltpu.VMEM((K, CHUNK), jnp.int32),
            pltpu.VMEM((NBUF, CHUNK, LANES), jnp.float32),
            pltpu.VMEM((RPS, LANES), jnp.float32),
            pltpu.VMEM((RPF, 128), jnp.float32),
            pltpu.VMEM_SHARED((NPAD, LANES), jnp.float32),
            pltpu.VMEM_SHARED((NPAD, LANES), jnp.float32),
            pltpu.SemaphoreType.DMA((NBUF,)),
            pltpu.SemaphoreType.DMA((NBUF,)),
        ],
        compiler_params=cp,
    )
    return deg, agg


_tc_mm = pl.pallas_call(
    _tc_mm_body, out_shape=jax.ShapeDtypeStruct((NROWS, 128), jnp.float32))
_tc_scale = pl.pallas_call(
    _tc_scale_body, out_shape=jax.ShapeDtypeStruct((NROWS, 128), jnp.float32))
_tc2 = pl.pallas_call(
    _tc2_body, out_shape=jax.ShapeDtypeStruct((NROWS, 128), jnp.float32))
_tc3 = pl.pallas_call(
    _tc3_body, out_shape=jax.ShapeDtypeStruct((NROWS, 8 * OUT), jnp.float32))


def kernel(features, edge_index, W1, b1, W2, b2):
    srcs = edge_index[0].reshape(NW, K, CHUNK)
    dsts = edge_index[1].reshape(NW, K, CHUNK)
    feat_flat = jnp.concatenate(
        [features, jnp.zeros((NPAD - N_NODES, F_IN), jnp.float32)]
    ).reshape(NROWS, 8 * F_IN)
    eye8 = jnp.eye(8, dtype=jnp.float32)
    wk1 = jnp.kron(eye8, W1)
    wk2 = jnp.kron(eye8, W2)
    b1t = jnp.tile(b1, 8).reshape(1, 128)
    b2t = jnp.tile(b2, 8).reshape(1, 8 * OUT)
    zeros_rows = jnp.zeros((NPAD, LANES), jnp.float32)
    ones_rows = jnp.ones((CHUNK, LANES), jnp.float32)

    deg_kernel, agg_kernel = _sc_kernels()
    degs = deg_kernel(srcs, dsts, ones_rows, zeros_rows)
    h1p = _tc_mm(feat_flat, wk1)
    h1 = _tc_scale(h1p, degs)
    agg1 = agg_kernel(h1, srcs, dsts, zeros_rows)
    x2 = _tc2(agg1, degs, b1t)
    agg2 = agg_kernel(x2, srcs, dsts, zeros_rows)
    outg = _tc3(agg2, degs, wk2, b2t)
    return outg.reshape(NPAD, OUT)[:N_NODES]


# speedup vs baseline: 26.4202x; 1.0000x over previous
import functools

import jax
import jax.numpy as jnp
from jax import lax
from jax.experimental import pallas as pl
from jax.experimental.pallas import tpu as pltpu
from jax.experimental.pallas import tpu_sc as plsc

N_NODES = 10000
N_EDGES = 320000
F_IN = 128
HID = 16
OUT = 40

NC = 2
NS = 16
NW = NC * NS
LANES = 16
CHUNK = 128
EPT = N_EDGES // NW
NVEC = EPT // LANES
K = 80
NBUF = 8
NPAD = 10112
RPS = NPAD // NS
NROWS = NPAD // 8
RPF = NROWS // NS


def _repack_to_flat(buf16, buf128):
    @pl.loop(0, RPF)
    def _(r):
        for m in range(8):
            buf128.at[r, pl.ds(16 * m, 16)][...] = buf16.at[8 * r + m][...]


def _repack_from_flat(buf128, buf16):
    @pl.loop(0, RPF)
    def _(r):
        for m in range(8):
            buf16.at[8 * r + m][...] = buf128.at[r, pl.ds(16 * m, 16)][...]


def _repack_idx(idx1d, idx2d):
    @pl.loop(0, NVEC // 8)
    def _(j):
        for m in range(8):
            idx2d.at[j, pl.ds(16 * m, 16)][...] = \
                idx1d.at[pl.ds(16 * (8 * j + m), 16)][...]

    idx2d.at[NVEC // 8, pl.ds(0, 16)][...] = idx1d.at[pl.ds(EPT - 16, 16)][...]
    dummy = jnp.full((16,), N_NODES, jnp.int32)
    for m in range(1, 8):
        idx2d.at[NVEC // 8, pl.ds(16 * m, 16)][...] = dummy

    @pl.loop(NVEC // 8 + 1, K)
    def _(j):
        for m in range(8):
            idx2d.at[j, pl.ds(16 * m, 16)][...] = dummy


def _deg_body(edges_hbm, ones_hbm, zeros_hbm, out_hbm, idx_hbm,
              idx1d_v, sidx_v, didx_v, ones_v, buf16, buf128, ds_sh, dd_sh,
              ssem, dsem, wsem):
    c = lax.axis_index("c")
    s = lax.axis_index("s")
    wid = c * NS + s
    r0 = s * RPS
    f0 = s * RPF
    pltpu.sync_copy(zeros_hbm.at[pl.ds(r0, RPS)], ds_sh.at[pl.ds(r0, RPS)])
    pltpu.sync_copy(zeros_hbm.at[pl.ds(r0, RPS)], dd_sh.at[pl.ds(r0, RPS)])
    pltpu.sync_copy(ones_hbm, ones_v)
    pltpu.sync_copy(edges_hbm.at[0, pl.ds(wid * EPT, EPT)], idx1d_v)
    _repack_idx(idx1d_v, sidx_v)
    pltpu.sync_copy(edges_hbm.at[1, pl.ds(wid * EPT, EPT)], idx1d_v)
    _repack_idx(idx1d_v, didx_v)
    pltpu.async_copy(sidx_v, idx_hbm.at[0, wid], wsem.at[0])
    pltpu.async_copy(didx_v, idx_hbm.at[1, wid], wsem.at[1])
    plsc.subcore_barrier()

    @pl.loop(0, K)
    def _(j):
        pltpu.async_copy(ones_v, ds_sh.at[sidx_v.at[j]], ssem, add=True)
        pltpu.async_copy(ones_v, dd_sh.at[didx_v.at[j]], dsem, add=True)

    pltpu.make_async_copy(sidx_v, idx_hbm.at[0, wid], wsem.at[0]).wait()
    pltpu.make_async_copy(didx_v, idx_hbm.at[1, wid], wsem.at[1]).wait()

    @pl.loop(0, K)
    def _(j):
        pltpu.make_async_copy(ones_v, ds_sh.at[sidx_v.at[j]], ssem).wait()
        pltpu.make_async_copy(ones_v, dd_sh.at[didx_v.at[j]], dsem).wait()

    plsc.subcore_barrier()
    pltpu.sync_copy(ds_sh.at[pl.ds(r0, RPS)], buf16)
    _repack_to_flat(buf16, buf128)
    pltpu.sync_copy(buf128, out_hbm.at[c, 0, pl.ds(f0, RPF)])
    pltpu.sync_copy(dd_sh.at[pl.ds(r0, RPS)], buf16)
    _repack_to_flat(buf16, buf128)
    pltpu.sync_copy(buf128, out_hbm.at[c, 1, pl.ds(f0, RPF)])


def _agg_body(h_hbm, idx_hbm, zeros_hbm, out_hbm,
              sidx_v, didx_v, rows_v, buf16, buf128, h_sh, acc_sh,
              gsem, ssem):
    c = lax.axis_index("c")
    s = lax.axis_index("s")
    wid = c * NS + s
    r0 = s * RPS
    f0 = s * RPF
    pltpu.sync_copy(zeros_hbm.at[pl.ds(r0, RPS)], acc_sh.at[pl.ds(r0, RPS)])
    pltpu.sync_copy(h_hbm.at[pl.ds(f0, RPF)], buf128)
    _repack_from_flat(buf128, buf16)
    pltpu.sync_copy(buf16, h_sh.at[pl.ds(r0, RPS)])
    pltpu.sync_copy(idx_hbm.at[0, wid], sidx_v)
    pltpu.sync_copy(idx_hbm.at[1, wid], didx_v)
    plsc.subcore_barrier()

    for b in range(NBUF):
        pltpu.async_copy(h_sh.at[sidx_v.at[b]], rows_v.at[b], gsem.at[b])

    @pl.loop(0, K // NBUF)
    def _(i):
        j0 = i * NBUF
        for b in range(NBUF):
            pltpu.make_async_copy(
                h_sh.at[sidx_v.at[j0 + b]], rows_v.at[b], gsem.at[b]).wait()
            pltpu.async_copy(
                rows_v.at[b], acc_sh.at[didx_v.at[j0 + b]], ssem.at[b],
                add=True)

        @pl.when(i < K // NBUF - 1)
        def _():
            for b in range(NBUF):
                pltpu.make_async_copy(
                    rows_v.at[b], acc_sh.at[didx_v.at[j0 + b]],
                    ssem.at[b]).wait()
                pltpu.async_copy(
                    h_sh.at[sidx_v.at[j0 + NBUF + b]], rows_v.at[b],
                    gsem.at[b])

    for b in range(NBUF):
        pltpu.make_async_copy(
            rows_v.at[b], acc_sh.at[didx_v.at[K - NBUF + b]], ssem.at[b]).wait()

    plsc.subcore_barrier()
    pltpu.sync_copy(acc_sh.at[pl.ds(r0, RPS)], buf16)
    _repack_to_flat(buf16, buf128)
    pltpu.sync_copy(buf128, out_hbm.at[c, pl.ds(f0, RPF)])


def _tc_mm_body(feat_ref, wk1_ref, out_ref):
    out_ref[...] = jnp.dot(feat_ref[...], wk1_ref[...],
                           preferred_element_type=jnp.float32)


def _tc_scale_body(h_ref, degs_ref, out_ref):
    degs = degs_ref[...]
    norm_src = lax.rsqrt(jnp.maximum(degs[0, 0] + degs[1, 0], 1.0))
    out_ref[...] = h_ref[...] * norm_src


def _tc2_body(aggs_ref, degs_ref, b1_ref, out_ref):
    degs = degs_ref[...]
    norm_src = lax.rsqrt(jnp.maximum(degs[0, 0] + degs[1, 0], 1.0))
    norm_dst = lax.rsqrt(jnp.maximum(degs[0, 1] + degs[1, 1], 1.0))
    aggs = aggs_ref[...]
    agg = aggs[0] + aggs[1]
    out_ref[...] = jnp.maximum(agg * norm_dst + b1_ref[...], 0.0) * norm_src


def _tc3_body(aggs_ref, degs_ref, wk2_ref, b2_ref, out_ref):
    degs = degs_ref[...]
    norm_dst = lax.rsqrt(jnp.maximum(degs[0, 1] + degs[1, 1], 1.0))
    aggs = aggs_ref[...]
    scaled = (aggs[0] + aggs[1]) * norm_dst
    rst = jnp.dot(scaled, wk2_ref[...], preferred_element_type=jnp.float32)
    out_ref[...] = rst + b2_ref[...]


@functools.cache
def _sc_kernels():
    mesh = plsc.VectorSubcoreMesh(core_axis_name="c", subcore_axis_name="s",
                                  num_cores=NC, num_subcores=NS)
    cp = pltpu.CompilerParams(use_tc_tiling_on_sc=False)
    deg = pl.kernel(
        _deg_body,
        out_type=(
            jax.ShapeDtypeStruct((NC, 2, NROWS, 128), jnp.float32),
            jax.ShapeDtypeStruct((2, NW, K, CHUNK), jnp.int32),
        ),
        mesh=mesh,
        scratch_types=[
            pltpu.VMEM((EPT,), jnp.int32),
            pltpu.VMEM((K, CHUNK), jnp.int32),
            pltpu.VMEM((K, CHUNK), jnp.int32),
            pltpu.VMEM((CHUNK, LANES), jnp.float32),
            pltpu.VMEM((RPS, LANES), jnp.float32),
            pltpu.VMEM((RPF, 128), jnp.float32),
            pltpu.VMEM_SHARED((NPAD, LANES), jnp.float32),
            pltpu.VMEM_SHARED((NPAD, LANES), jnp.float32),
            pltpu.SemaphoreType.DMA,
            pltpu.SemaphoreType.DMA,
            pltpu.SemaphoreType.DMA((2,)),
        ],
        compiler_params=cp,
    )
    agg = pl.kernel(
        _agg_body,
        out_type=jax.ShapeDtypeStruct((NC, NROWS, 128), jnp.float32),
        mesh=mesh,
        scratch_types=[
            pltpu.VMEM((K, CHUNK), jnp.int32),
            pltpu.VMEM((K, CHUNK), jnp.int32),
            pltpu.VMEM((NBUF, CHUNK, LANES), jnp.float32),
            pltpu.VMEM((RPS, LANES), jnp.float32),
            pltpu.VMEM((RPF, 128), jnp.float32),
            pltpu.VMEM_SHARED((NPAD, LANES), jnp.float32),
            pltpu.VMEM_SHARED((NPAD, LANES), jnp.float32),
            pltpu.SemaphoreType.DMA((NBUF,)),
            pltpu.SemaphoreType.DMA((NBUF,)),
        ],
        compiler_params=cp,
    )
    return deg, agg


_tc_mm = pl.pallas_call(
    _tc_mm_body, out_shape=jax.ShapeDtypeStruct((NROWS, 128), jnp.float32))
_tc_scale = pl.pallas_call(
    _tc_scale_body, out_shape=jax.ShapeDtypeStruct((NROWS, 128), jnp.float32))
_tc2 = pl.pallas_call(
    _tc2_body, out_shape=jax.ShapeDtypeStruct((NROWS, 128), jnp.float32))
_tc3 = pl.pallas_call(
    _tc3_body, out_shape=jax.ShapeDtypeStruct((NROWS, 8 * OUT), jnp.float32))


def kernel(features, edge_index, W1, b1, W2, b2):
    feat_flat = jnp.concatenate(
        [features, jnp.zeros((NPAD - N_NODES, F_IN), jnp.float32)]
    ).reshape(NROWS, 8 * F_IN)
    eye8 = jnp.eye(8, dtype=jnp.float32)
    wk1 = jnp.kron(eye8, W1)
    wk2 = jnp.kron(eye8, W2)
    b1t = jnp.tile(b1, 8).reshape(1, 128)
    b2t = jnp.tile(b2, 8).reshape(1, 8 * OUT)
    zeros_rows = jnp.zeros((NPAD, LANES), jnp.float32)
    ones_rows = jnp.ones((CHUNK, LANES), jnp.float32)

    deg_kernel, agg_kernel = _sc_kernels()
    degs, idx2 = deg_kernel(edge_index, ones_rows, zeros_rows)
    h1p = _tc_mm(feat_flat, wk1)
    h1 = _tc_scale(h1p, degs)
    agg1 = agg_kernel(h1, idx2, zeros_rows)
    x2 = _tc2(agg1, degs, b1t)
    agg2 = agg_kernel(x2, idx2, zeros_rows)
    outg = _tc3(agg2, degs, wk2, b2t)
    return outg.reshape(NPAD, OUT)[:N_NODES]


# speedup vs baseline: 27.3802x; 1.0363x over previous
import functools

import jax
import jax.numpy as jnp
from jax import lax
from jax.experimental import pallas as pl
from jax.experimental.pallas import tpu as pltpu
from jax.experimental.pallas import tpu_sc as plsc

N_NODES = 10000
N_EDGES = 320000
F_IN = 128
HID = 16
OUT = 40

NC = 2
NS = 16
NW = NC * NS
LANES = 16
CHUNK = 128
EPT = N_EDGES // NW
NVEC = EPT // LANES
K = 80
NBUF = 8
NPAD = 10112
RPS = NPAD // NS
NROWS = NPAD // 8
RPF = NROWS // NS


def _repack_to_flat(buf16, buf128):
    @pl.loop(0, RPF)
    def _(r):
        for m in range(8):
            buf128.at[r, pl.ds(16 * m, 16)][...] = buf16.at[8 * r + m][...]


def _repack_from_flat(buf128, buf16):
    @pl.loop(0, RPF)
    def _(r):
        for m in range(8):
            buf16.at[8 * r + m][...] = buf128.at[r, pl.ds(16 * m, 16)][...]


def _repack_idx(idx1d, idx2d):
    @pl.loop(0, NVEC // 8)
    def _(j):
        for m in range(8):
            idx2d.at[j, pl.ds(16 * m, 16)][...] = \
                idx1d.at[pl.ds(16 * (8 * j + m), 16)][...]

    idx2d.at[NVEC // 8, pl.ds(0, 16)][...] = idx1d.at[pl.ds(EPT - 16, 16)][...]
    dummy = jnp.full((16,), N_NODES, jnp.int32)
    for m in range(1, 8):
        idx2d.at[NVEC // 8, pl.ds(16 * m, 16)][...] = dummy

    @pl.loop(NVEC // 8 + 1, K)
    def _(j):
        for m in range(8):
            idx2d.at[j, pl.ds(16 * m, 16)][...] = dummy


def _deg_body(edges_hbm, ones_hbm, zeros_hbm, out_hbm, idx_hbm,
              idx1d_v, sidx_v, didx_v, ones_v, buf16, buf128, ds_sh, dd_sh,
              ssem, dsem, wsem, isem):
    c = lax.axis_index("c")
    s = lax.axis_index("s")
    wid = c * NS + s
    r0 = s * RPS
    f0 = s * RPF
    pltpu.async_copy(edges_hbm.at[0, pl.ds(wid * EPT, EPT)], idx1d_v,
                     isem.at[0])
    pltpu.async_copy(zeros_hbm.at[pl.ds(r0, RPS)], ds_sh.at[pl.ds(r0, RPS)],
                     isem.at[1])
    pltpu.async_copy(zeros_hbm.at[pl.ds(r0, RPS)], dd_sh.at[pl.ds(r0, RPS)],
                     isem.at[2])
    pltpu.async_copy(ones_hbm, ones_v, isem.at[3])
    pltpu.make_async_copy(edges_hbm.at[0, pl.ds(wid * EPT, EPT)], idx1d_v,
                          isem.at[0]).wait()
    _repack_idx(idx1d_v, sidx_v)
    pltpu.async_copy(sidx_v, idx_hbm.at[0, wid], wsem.at[0])
    pltpu.make_async_copy(zeros_hbm.at[pl.ds(r0, RPS)],
                          ds_sh.at[pl.ds(r0, RPS)], isem.at[1]).wait()
    pltpu.make_async_copy(zeros_hbm.at[pl.ds(r0, RPS)],
                          dd_sh.at[pl.ds(r0, RPS)], isem.at[2]).wait()
    pltpu.make_async_copy(ones_hbm, ones_v, isem.at[3]).wait()
    plsc.subcore_barrier()

    @pl.loop(0, K)
    def _(j):
        pltpu.async_copy(ones_v, ds_sh.at[sidx_v.at[j]], ssem, add=True)

    pltpu.sync_copy(edges_hbm.at[1, pl.ds(wid * EPT, EPT)], idx1d_v)
    _repack_idx(idx1d_v, didx_v)
    pltpu.async_copy(didx_v, idx_hbm.at[1, wid], wsem.at[1])

    @pl.loop(0, K)
    def _(j):
        pltpu.async_copy(ones_v, dd_sh.at[didx_v.at[j]], dsem, add=True)

    pltpu.make_async_copy(sidx_v, idx_hbm.at[0, wid], wsem.at[0]).wait()
    pltpu.make_async_copy(didx_v, idx_hbm.at[1, wid], wsem.at[1]).wait()

    @pl.loop(0, K)
    def _(j):
        pltpu.make_async_copy(ones_v, ds_sh.at[sidx_v.at[j]], ssem).wait()
        pltpu.make_async_copy(ones_v, dd_sh.at[didx_v.at[j]], dsem).wait()

    plsc.subcore_barrier()
    pltpu.sync_copy(ds_sh.at[pl.ds(r0, RPS)], buf16)
    _repack_to_flat(buf16, buf128)
    pltpu.sync_copy(buf128, out_hbm.at[c, 0, pl.ds(f0, RPF)])
    pltpu.sync_copy(dd_sh.at[pl.ds(r0, RPS)], buf16)
    _repack_to_flat(buf16, buf128)
    pltpu.sync_copy(buf128, out_hbm.at[c, 1, pl.ds(f0, RPF)])


def _agg_body(h_hbm, idx_hbm, zeros_hbm, out_hbm,
              sidx_v, didx_v, rows_v, buf16, buf128, h_sh, acc_sh,
              gsem, ssem, isem):
    c = lax.axis_index("c")
    s = lax.axis_index("s")
    wid = c * NS + s
    r0 = s * RPS
    f0 = s * RPF
    pltpu.async_copy(h_hbm.at[pl.ds(f0, RPF)], buf128, isem.at[0])
    pltpu.async_copy(zeros_hbm.at[pl.ds(r0, RPS)], acc_sh.at[pl.ds(r0, RPS)],
                     isem.at[1])
    pltpu.async_copy(idx_hbm.at[0, wid], sidx_v, isem.at[2])
    pltpu.async_copy(idx_hbm.at[1, wid], didx_v, isem.at[3])
    pltpu.make_async_copy(h_hbm.at[pl.ds(f0, RPF)], buf128, isem.at[0]).wait()
    _repack_from_flat(buf128, buf16)
    pltpu.sync_copy(buf16, h_sh.at[pl.ds(r0, RPS)])
    pltpu.make_async_copy(zeros_hbm.at[pl.ds(r0, RPS)],
                          acc_sh.at[pl.ds(r0, RPS)], isem.at[1]).wait()
    pltpu.make_async_copy(idx_hbm.at[0, wid], sidx_v, isem.at[2]).wait()
    pltpu.make_async_copy(idx_hbm.at[1, wid], didx_v, isem.at[3]).wait()
    plsc.subcore_barrier()

    for b in range(NBUF):
        pltpu.async_copy(h_sh.at[sidx_v.at[b]], rows_v.at[b], gsem.at[b])

    @pl.loop(0, K // NBUF)
    def _(i):
        j0 = i * NBUF
        for b in range(NBUF):
            pltpu.make_async_copy(
                h_sh.at[sidx_v.at[j0 + b]], rows_v.at[b], gsem.at[b]).wait()
            pltpu.async_copy(
                rows_v.at[b], acc_sh.at[didx_v.at[j0 + b]], ssem.at[b],
                add=True)

        @pl.when(i < K // NBUF - 1)
        def _():
            for b in range(NBUF):
                pltpu.make_async_copy(
                    rows_v.at[b], acc_sh.at[didx_v.at[j0 + b]],
                    ssem.at[b]).wait()
                pltpu.async_copy(
                    h_sh.at[sidx_v.at[j0 + NBUF + b]], rows_v.at[b],
                    gsem.at[b])

    for b in range(NBUF):
        pltpu.make_async_copy(
            rows_v.at[b], acc_sh.at[didx_v.at[K - NBUF + b]], ssem.at[b]).wait()

    plsc.subcore_barrier()
    pltpu.sync_copy(acc_sh.at[pl.ds(r0, RPS)], buf16)
    _repack_to_flat(buf16, buf128)
    pltpu.sync_copy(buf128, out_hbm.at[c, pl.ds(f0, RPF)])


def _tc_mm_body(feat_ref, wk1_ref, out_ref):
    out_ref[...] = jnp.dot(feat_ref[...], wk1_ref[...],
                           preferred_element_type=jnp.float32)


def _tc_scale_body(h_ref, degs_ref, out_ref):
    degs = degs_ref[...]
    norm_src = lax.rsqrt(jnp.maximum(degs[0, 0] + degs[1, 0], 1.0))
    out_ref[...] = h_ref[...] * norm_src


def _tc2_body(aggs_ref, degs_ref, b1_ref, out_ref):
    degs = degs_ref[...]
    norm_src = lax.rsqrt(jnp.maximum(degs[0, 0] + degs[1, 0], 1.0))
    norm_dst = lax.rsqrt(jnp.maximum(degs[0, 1] + degs[1, 1], 1.0))
    aggs = aggs_ref[...]
    agg = aggs[0] + aggs[1]
    out_ref[...] = jnp.maximum(agg * norm_dst + b1_ref[...], 0.0) * norm_src


def _tc3_body(aggs_ref, degs_ref, wk2_ref, b2_ref, out_ref):
    degs = degs_ref[...]
    norm_dst = lax.rsqrt(jnp.maximum(degs[0, 1] + degs[1, 1], 1.0))
    aggs = aggs_ref[...]
    scaled = (aggs[0] + aggs[1]) * norm_dst
    rst = jnp.dot(scaled, wk2_ref[...], preferred_element_type=jnp.float32)
    out_ref[...] = rst + b2_ref[...]


@functools.cache
def _sc_kernels():
    mesh = plsc.VectorSubcoreMesh(core_axis_name="c", subcore_axis_name="s",
                                  num_cores=NC, num_subcores=NS)
    cp = pltpu.CompilerParams(use_tc_tiling_on_sc=False)
    deg = pl.kernel(
        _deg_body,
        out_type=(
            jax.ShapeDtypeStruct((NC, 2, NROWS, 128), jnp.float32),
            jax.ShapeDtypeStruct((2, NW, K, CHUNK), jnp.int32),
        ),
        mesh=mesh,
        scratch_types=[
            pltpu.VMEM((EPT,), jnp.int32),
            pltpu.VMEM((K, CHUNK), jnp.int32),
            pltpu.VMEM((K, CHUNK), jnp.int32),
            pltpu.VMEM((CHUNK, LANES), jnp.float32),
            pltpu.VMEM((RPS, LANES), jnp.float32),
            pltpu.VMEM((RPF, 128), jnp.float32),
            pltpu.VMEM_SHARED((NPAD, LANES), jnp.float32),
            pltpu.VMEM_SHARED((NPAD, LANES), jnp.float32),
            pltpu.SemaphoreType.DMA,
            pltpu.SemaphoreType.DMA,
            pltpu.SemaphoreType.DMA((2,)),
            pltpu.SemaphoreType.DMA((4,)),
        ],
        compiler_params=cp,
    )
    agg = pl.kernel(
        _agg_body,
        out_type=jax.ShapeDtypeStruct((NC, NROWS, 128), jnp.float32),
        mesh=mesh,
        scratch_types=[
            pltpu.VMEM((K, CHUNK), jnp.int32),
            pltpu.VMEM((K, CHUNK), jnp.int32),
            pltpu.VMEM((NBUF, CHUNK, LANES), jnp.float32),
            pltpu.VMEM((RPS, LANES), jnp.float32),
            pltpu.VMEM((RPF, 128), jnp.float32),
            pltpu.VMEM_SHARED((NPAD, LANES), jnp.float32),
            pltpu.VMEM_SHARED((NPAD, LANES), jnp.float32),
            pltpu.SemaphoreType.DMA((NBUF,)),
            pltpu.SemaphoreType.DMA((NBUF,)),
            pltpu.SemaphoreType.DMA((4,)),
        ],
        compiler_params=cp,
    )
    return deg, agg


_tc_mm = pl.pallas_call(
    _tc_mm_body, out_shape=jax.ShapeDtypeStruct((NROWS, 128), jnp.float32))
_tc_scale = pl.pallas_call(
    _tc_scale_body, out_shape=jax.ShapeDtypeStruct((NROWS, 128), jnp.float32))
_tc2 = pl.pallas_call(
    _tc2_body, out_shape=jax.ShapeDtypeStruct((NROWS, 128), jnp.float32))
_tc3 = pl.pallas_call(
    _tc3_body, out_shape=jax.ShapeDtypeStruct((NROWS, 8 * OUT), jnp.float32))


def kernel(features, edge_index, W1, b1, W2, b2):
    feat_flat = jnp.concatenate(
        [features, jnp.zeros((NPAD - N_NODES, F_IN), jnp.float32)]
    ).reshape(NROWS, 8 * F_IN)
    eye8 = jnp.eye(8, dtype=jnp.float32)
    wk1 = jnp.kron(eye8, W1)
    wk2 = jnp.kron(eye8, W2)
    b1t = jnp.tile(b1, 8).reshape(1, 128)
    b2t = jnp.tile(b2, 8).reshape(1, 8 * OUT)
    zeros_rows = jnp.zeros((NPAD, LANES), jnp.float32)
    ones_rows = jnp.ones((CHUNK, LANES), jnp.float32)

    deg_kernel, agg_kernel = _sc_kernels()
    degs, idx2 = deg_kernel(edge_index, ones_rows, zeros_rows)
    h1p = _tc_mm(feat_flat, wk1)
    h1 = _tc_scale(h1p, degs)
    agg1 = agg_kernel(h1, idx2, zeros_rows)
    x2 = _tc2(agg1, degs, b1t)
    agg2 = agg_kernel(x2, idx2, zeros_rows)
    outg = _tc3(agg2, degs, wk2, b2t)
    return outg[:N_NODES // 8].reshape(N_NODES, OUT)
